# baseline (speedup 1.0000x reference)
import jax
import jax.numpy as jnp
from jax import lax
from jax.experimental import pallas as pl
from jax.experimental.pallas import tpu as pltpu
from jax.experimental.pallas import tpu_sc as plsc

B = 16
T = 2048
D = 64
VOCAB = 1000000
NC = 2
NS = 16
NW = NC * NS
NB = 8
TW = 128
LANES = 16
NPASS = 4
BPP = NB // NPASS
SPP = BPP * TW
GPP = SPP // LANES


def _emb_body(x_hbm, wte2_hbm, wpe2_hbm, out2_hbm,
              idx_v, row_v, off_v, rows2_v, stage2_v, sem):
    wid = lax.axis_index("s") * NC + lax.axis_index("c")
    b0 = pl.multiple_of((wid % 2) * NB, NB)
    t0 = pl.multiple_of((wid // 2) * TW, TW)

    pltpu.sync_copy(x_hbm.at[pl.ds(b0, NB), pl.ds(t0, TW)], idx_v)

    def split_idx(k, carry):
        b = k // (TW // LANES)
        j = (k % (TW // LANES)) * LANES
        ix = idx_v[b, pl.ds(j, LANES)]
        row_v[b, pl.ds(j, LANES)] = lax.shift_right_logical(ix, 1)
        off_v[b, pl.ds(j, LANES)] = lax.shift_left(jnp.bitwise_and(ix, 1), 6)
        return carry

    lax.fori_loop(0, NB * TW // LANES, split_idx, 0)

    iota = lax.iota(jnp.int32, LANES)

    for p in range(NPASS):
        copies = []
        for bl in range(BPP):
            b = p * BPP + bl
            copies.append(pltpu.async_copy(
                wte2_hbm.at[row_v.at[b]],
                rows2_v.at[pl.ds(bl * TW, TW)], sem))
            copies.append(pltpu.async_copy(
                wpe2_hbm.at[pl.ds(pl.multiple_of(t0 * D // 128, 64), TW * D // 128)],
                stage2_v.at[pl.ds(bl * TW * D // 128, TW * D // 128)], sem))
        for cp in copies:
            cp.wait()

        def select_g(g, carry):
            s_vec = g * LANES + iota
            b = p * BPP + g // (TW // LANES)
            j0 = (g % (TW // LANES)) * LANES
            off = off_v[b, pl.ds(j0, LANES)]
            r2 = lax.shift_right_logical(s_vec, 1)
            c2 = lax.shift_left(jnp.bitwise_and(s_vec, 1), 6)
            for c in range(D):
                val = plsc.load_gather(rows2_v, [s_vec, off + c])
                plsc.addupdate_scatter(stage2_v, [r2, c2 + c], val)
            return carry

        lax.fori_loop(0, GPP, select_g, 0)

        outs = [
            pltpu.async_copy(
                stage2_v.at[pl.ds(bl * TW * D // 128, TW * D // 128)],
                out2_hbm.at[pl.ds(
                    pl.multiple_of(((b0 + p * BPP + bl) * T + t0) * D // 128, 64),
                    TW * D // 128)], sem)
            for bl in range(BPP)
        ]
        for o in outs:
            o.wait()


@jax.jit
def kernel(x, wte, wpe):
    wte2 = wte.reshape(VOCAB // 2, 2 * D)
    wpe2 = wpe.reshape(T * D // 128, 128)
    run = pl.kernel(
        _emb_body,
        out_type=jax.ShapeDtypeStruct((B * T * D // 128, 128), jnp.float32),
        mesh=plsc.VectorSubcoreMesh(core_axis_name="c", subcore_axis_name="s"),
        scratch_types=[
            pltpu.VMEM((NB, TW), jnp.int32),
            pltpu.VMEM((NB, TW), jnp.int32),
            pltpu.VMEM((NB, TW), jnp.int32),
            pltpu.VMEM((SPP, 2 * D), jnp.float32),
            pltpu.VMEM((SPP * D // 128, 128), jnp.float32),
            pltpu.SemaphoreType.DMA,
        ],
        compiler_params=pltpu.CompilerParams(needs_layout_passes=False),
    )
    out2 = run(x, wte2, wpe2)
    return out2.reshape(B, T, D)


# speedup vs baseline: 1.5940x; 1.5940x over previous
import jax
import jax.numpy as jnp
from jax import lax
from jax.experimental import pallas as pl
from jax.experimental.pallas import tpu as pltpu
from jax.experimental.pallas import tpu_sc as plsc

B = 16
T = 2048
D = 64
VOCAB = 1000000
NC = 2
NS = 16
NW = NC * NS
NB = 8
TW = 128
LANES = 16
VPD = D // LANES
NCHUNK = TW // LANES


def _emb_body(x_hbm, wte_hbm, wpe2_hbm, out2_hbm,
              idx_v, idx_s, slab_v, stage_v, sem_a, sem_b):
    wid = lax.axis_index("s") * NC + lax.axis_index("c")
    b0 = pl.multiple_of((wid % 2) * NB, NB)
    t0 = pl.multiple_of((wid // 2) * TW, TW)

    pltpu.sync_copy(x_hbm.at[pl.ds(b0, NB), pl.ds(t0, TW)], idx_v)

    iota = lax.iota(jnp.int32, LANES)

    def extract(k, carry):
        b = k // NCHUNK
        j = (k % NCHUNK) * LANES
        ix_vec = idx_v[b, pl.ds(j, LANES)]
        for l in range(LANES):
            s = jnp.sum(jnp.where(iota == l, ix_vec, 0))
            idx_s[k * LANES + l] = s
        return carry

    lax.fori_loop(0, NB * NCHUNK, extract, 0)

    sems = (sem_a, sem_b)

    def fire_chunk(b, c):
        base = b * TW + c * LANES
        par = c % 2
        descs = []
        for i in range(LANES):
            ix = idx_s[base + i]
            slab8 = pl.multiple_of(lax.shift_right_logical(ix, 3) * 8, 8)
            descs.append(pltpu.async_copy(
                wte_hbm.at[pl.ds(slab8, 8), :],
                slab_v.at[par * LANES + i], sems[par]))
        return descs

    def run_b(b, carry):
        pltpu.sync_copy(
            wpe2_hbm.at[pl.ds(pl.multiple_of(t0 * D // 128, 64), TW * D // 128)],
            stage_v)

        descs = fire_chunk(b, 0)
        for c in range(NCHUNK):
            nxt = fire_chunk(b, c + 1) if c + 1 < NCHUNK else []
            for d in descs:
                d.wait()
            descs = nxt
            par = c % 2
            base = b * TW + c * LANES
            for i in range(LANES):
                ix = idx_s[base + i]
                row = jnp.bitwise_and(ix, 7)
                jj = c * LANES + i
                r2 = jj // 2
                c2 = (jj % 2) * D
                for v in range(VPD):
                    val = slab_v[par * LANES + i, row, pl.ds(v * LANES, LANES)]
                    plsc.addupdate(
                        stage_v.at[r2, pl.ds(c2 + v * LANES, LANES)], val)

        out_off = pl.multiple_of(((b0 + b) * T + t0) * D // 128, 64)
        pltpu.sync_copy(stage_v, out2_hbm.at[pl.ds(out_off, TW * D // 128)])
        return carry

    lax.fori_loop(0, NB, run_b, 0)


@jax.jit
def kernel(x, wte, wpe):
    wpe2 = wpe.reshape(T * D // 128, 128)
    run = pl.kernel(
        _emb_body,
        out_type=jax.ShapeDtypeStruct((B * T * D // 128, 128), jnp.float32),
        mesh=plsc.VectorSubcoreMesh(core_axis_name="c", subcore_axis_name="s"),
        scratch_types=[
            pltpu.VMEM((NB, TW), jnp.int32),
            pltpu.SMEM((NB * TW,), jnp.int32),
            pltpu.VMEM((2 * LANES, 8, D), jnp.float32),
            pltpu.VMEM((TW * D // 128, 128), jnp.float32),
            pltpu.SemaphoreType.DMA,
            pltpu.SemaphoreType.DMA,
        ],
        compiler_params=pltpu.CompilerParams(needs_layout_passes=False),
    )
    out2 = run(x, wte, wpe2)
    return out2.reshape(B, T, D)
